# baseline (speedup 1.0000x reference)
import jax
import jax.numpy as jnp
from jax import lax
from jax.experimental import pallas as pl
from jax.experimental.pallas import tpu as pltpu

B, N, D, R, E = 2, 1024, 128, 3, 8


def _moe_gat_kernel(x_ref, adj_ref, Wg_ref, bg_ref, Wr_ref, as_ref, ad_ref,
                    out_ref, gate_s, mask_s):
    r_idx = pl.program_id(1)
    e_idx = pl.program_id(2)

    @pl.when((r_idx == 0) & (e_idx == 0))
    def _init():
        out_ref[...] = jnp.zeros_like(out_ref)
        xg = x_ref[0]
        gl = jnp.dot(xg, Wg_ref[...], preferred_element_type=jnp.float32)
        gl = gl + bg_ref[...]
        gl = gl - jnp.max(gl, axis=1, keepdims=True)
        gexp = jnp.exp(gl)
        gate_s[...] = gexp / jnp.sum(gexp, axis=1, keepdims=True)

    @pl.when(e_idx == 0)
    def _mask():
        mask_s[...] = adj_ref[0, 0].astype(jnp.bfloat16)

    x = x_ref[0]
    W = Wr_ref[0, 0]
    h = jnp.dot(x, W, preferred_element_type=jnp.float32)

    asrc = as_ref[0]
    adst = ad_ref[0]
    es = jnp.sum(h * asrc, axis=1, keepdims=True)
    ed = lax.dot_general(adst, h, (((1,), (1,)), ((), ())),
                         preferred_element_type=jnp.float32)

    edmax = jnp.max(ed, axis=1, keepdims=True)
    se = es + edmax
    shift = jnp.maximum(se, 0.2 * se)
    a_pos = jnp.exp(se - shift).astype(jnp.bfloat16)
    a_neg = jnp.exp(0.2 * se - shift).astype(jnp.bfloat16)
    b_pos = jnp.exp(ed - edmax).astype(jnp.bfloat16)
    b_neg = jnp.exp(0.2 * (ed - edmax)).astype(jnp.bfloat16)

    pu = jnp.maximum(a_pos * b_pos, a_neg * b_neg)
    p = pu * mask_s[...]

    ones_col = jnp.ones((N, 1), dtype=jnp.bfloat16)
    denom = jnp.dot(p, ones_col,
                    preferred_element_type=jnp.float32)
    contrib = jnp.dot(p, h.astype(jnp.bfloat16),
                      preferred_element_type=jnp.float32)

    onehot = (lax.broadcasted_iota(jnp.int32, (1, E), 1) == e_idx)
    ge = jnp.sum(gate_s[...] * onehot.astype(jnp.float32), axis=1,
                 keepdims=True)

    scale = ge / jnp.maximum(denom, jnp.float32(1e-30))
    out_ref[0] = out_ref[0] + scale * contrib


def kernel(x, adj, Wg, bg, Wr, a_src, a_dst):
    bg2 = bg.reshape(1, E)
    grid = (B, R, E)
    out = pl.pallas_call(
        _moe_gat_kernel,
        grid=grid,
        in_specs=[
            pl.BlockSpec((1, N, D), lambda b, r, e: (b, 0, 0)),
            pl.BlockSpec((1, 1, N, N), lambda b, r, e: (b, r, 0, 0)),
            pl.BlockSpec((D, E), lambda b, r, e: (0, 0)),
            pl.BlockSpec((1, E), lambda b, r, e: (0, 0)),
            pl.BlockSpec((1, 1, D, D), lambda b, r, e: (e, r, 0, 0)),
            pl.BlockSpec((1, 1, D), lambda b, r, e: (e * R + r, 0, 0)),
            pl.BlockSpec((1, 1, D), lambda b, r, e: (e * R + r, 0, 0)),
        ],
        out_specs=pl.BlockSpec((1, N, D), lambda b, r, e: (b, 0, 0)),
        out_shape=jax.ShapeDtypeStruct((B, N, D), jnp.float32),
        scratch_shapes=[
            pltpu.VMEM((N, E), jnp.float32),
            pltpu.VMEM((N, N), jnp.bfloat16),
        ],
        compiler_params=pltpu.CompilerParams(
            dimension_semantics=("arbitrary", "arbitrary", "arbitrary"),
        ),
    )(x, adj, Wg, bg2, Wr,
      a_src.reshape(E * R, 1, D), a_dst.reshape(E * R, 1, D))
    return out


# speedup vs baseline: 1.1622x; 1.1622x over previous
import jax
import jax.numpy as jnp
from jax import lax
from jax.experimental import pallas as pl
from jax.experimental.pallas import tpu as pltpu

B, N, D, R, E = 2, 1024, 128, 3, 8


def _moe_gat_kernel(x_ref, adj_ref, Wg_ref, bg_ref, Wr_ref, as_ref, ad_ref,
                    out_ref, gate_s):
    r_idx = pl.program_id(1)

    @pl.when(r_idx == 0)
    def _init():
        out_ref[...] = jnp.zeros_like(out_ref)
        xg = x_ref[0]
        gl = jnp.dot(xg, Wg_ref[...], preferred_element_type=jnp.float32)
        gl = gl + bg_ref[...]
        gl = gl - jnp.max(gl, axis=1, keepdims=True)
        gexp = jnp.exp(gl)
        gate_s[...] = gexp / jnp.sum(gexp, axis=1, keepdims=True)

    x = x_ref[0]
    maskbf = adj_ref[0, 0].astype(jnp.bfloat16)
    asrc_all = as_ref[0]
    adst_all = ad_ref[0]
    ones_col = jnp.ones((N, 1), dtype=jnp.bfloat16)
    gate = gate_s[...]

    acc = jnp.zeros((N, D), jnp.float32)
    for e in range(E):
        W = Wr_ref[0, e]
        h = jnp.dot(x, W, preferred_element_type=jnp.float32)

        asrc = asrc_all[e:e + 1, :]
        adst = adst_all[e:e + 1, :]
        es = jnp.sum(h * asrc, axis=1, keepdims=True)
        ed = lax.dot_general(adst, h, (((1,), (1,)), ((), ())),
                             preferred_element_type=jnp.float32)

        edmax = jnp.max(ed, axis=1, keepdims=True)
        se = es + edmax
        shift = jnp.maximum(se, 0.2 * se)
        a_pos = jnp.exp(se - shift).astype(jnp.bfloat16)
        a_neg = jnp.exp(0.2 * se - shift).astype(jnp.bfloat16)
        b_pos = jnp.exp(ed - edmax).astype(jnp.bfloat16)
        b_neg = jnp.exp(0.2 * (ed - edmax)).astype(jnp.bfloat16)

        pu = jnp.maximum(a_pos * b_pos, a_neg * b_neg)
        p = pu * maskbf

        denom = jnp.dot(p, ones_col,
                        preferred_element_type=jnp.float32)
        contrib = jnp.dot(p, h.astype(jnp.bfloat16),
                          preferred_element_type=jnp.float32)

        ge = gate[:, e:e + 1]
        acc = acc + (ge / jnp.maximum(denom, jnp.float32(1e-30))) * contrib

    out_ref[0] = out_ref[0] + acc


def kernel(x, adj, Wg, bg, Wr, a_src, a_dst):
    bg2 = bg.reshape(1, E)
    WrT = jnp.transpose(Wr, (1, 0, 2, 3))
    asT = jnp.transpose(a_src, (1, 0, 2))
    adT = jnp.transpose(a_dst, (1, 0, 2))
    grid = (B, R)
    out = pl.pallas_call(
        _moe_gat_kernel,
        grid=grid,
        in_specs=[
            pl.BlockSpec((1, N, D), lambda b, r: (b, 0, 0)),
            pl.BlockSpec((1, 1, N, N), lambda b, r: (b, r, 0, 0)),
            pl.BlockSpec((D, E), lambda b, r: (0, 0)),
            pl.BlockSpec((1, E), lambda b, r: (0, 0)),
            pl.BlockSpec((1, E, D, D), lambda b, r: (r, 0, 0, 0)),
            pl.BlockSpec((1, E, D), lambda b, r: (r, 0, 0)),
            pl.BlockSpec((1, E, D), lambda b, r: (r, 0, 0)),
        ],
        out_specs=pl.BlockSpec((1, N, D), lambda b, r: (b, 0, 0)),
        out_shape=jax.ShapeDtypeStruct((B, N, D), jnp.float32),
        scratch_shapes=[
            pltpu.VMEM((N, E), jnp.float32),
        ],
        compiler_params=pltpu.CompilerParams(
            dimension_semantics=("arbitrary", "arbitrary"),
        ),
    )(x, adj, Wg, bg2, WrT, asT, adT)
    return out


# speedup vs baseline: 1.3682x; 1.1772x over previous
import jax
import jax.numpy as jnp
from jax import lax
from jax.experimental import pallas as pl
from jax.experimental.pallas import tpu as pltpu

B, N, D, R, E = 2, 1024, 128, 3, 8


def _moe_gat_kernel(x_ref, adj_ref, Wg_ref, bg_ref, Wr_ref, as_ref, ad_ref,
                    out_ref, gate_s):
    r_idx = pl.program_id(1)

    @pl.when(r_idx == 0)
    def _init():
        out_ref[...] = jnp.zeros_like(out_ref)
        xg = x_ref[0]
        gl = jnp.dot(xg, Wg_ref[...], preferred_element_type=jnp.float32)
        gl = gl + bg_ref[...]
        gl = gl - jnp.max(gl, axis=1, keepdims=True)
        gexp = jnp.exp(gl)
        gate_s[...] = gexp / jnp.sum(gexp, axis=1, keepdims=True)

    x_bf = x_ref[0].astype(jnp.bfloat16)
    maskbf = adj_ref[0, 0].astype(jnp.bfloat16)
    asrc_all = as_ref[0]
    adst_all = ad_ref[0]
    gate = gate_s[...]

    acc = jnp.zeros((N, D), jnp.float32)
    for e in range(E):
        W = Wr_ref[0, e]
        h = jnp.dot(x_bf, W,
                    preferred_element_type=jnp.float32).astype(jnp.bfloat16)

        asrc8 = asrc_all[8 * e:8 * e + 8, :]
        adst = adst_all[e:e + 1, :]
        es8 = lax.dot_general(h, asrc8, (((1,), (1,)), ((), ())),
                              preferred_element_type=jnp.float32)
        es = es8[:, 0:1]
        ed = lax.dot_general(adst, h, (((1,), (1,)), ((), ())),
                             preferred_element_type=jnp.float32)

        edmax = jnp.max(ed, axis=1, keepdims=True)
        se = es + edmax
        shift = jnp.maximum(se, 0.2 * se)
        a_pos = jnp.exp(se - shift).astype(jnp.bfloat16)
        a_neg = jnp.exp(0.2 * se - shift).astype(jnp.bfloat16)
        b_pos = jnp.exp(ed - edmax).astype(jnp.bfloat16)
        b_neg = jnp.exp(0.2 * (ed - edmax)).astype(jnp.bfloat16)

        pu = jnp.maximum(a_pos * b_pos, a_neg * b_neg)
        p = pu * maskbf

        denom = jnp.sum(p.astype(jnp.float32), axis=1,
                        keepdims=True)
        contrib = jnp.dot(p, h,
                          preferred_element_type=jnp.float32)

        ge = gate[:, e:e + 1]
        acc = acc + (ge / jnp.maximum(denom, jnp.float32(1e-30))) * contrib

    out_ref[0] = out_ref[0] + acc


def kernel(x, adj, Wg, bg, Wr, a_src, a_dst):
    bg2 = bg.reshape(1, E)
    WrT = jnp.transpose(Wr, (1, 0, 2, 3)).astype(jnp.bfloat16)
    asT = jnp.transpose(a_src, (1, 0, 2))
    as8 = jnp.zeros((R, 8 * E, D), jnp.float32)
    as8 = as8.at[:, ::8, :].set(asT).astype(jnp.bfloat16)
    adT = jnp.transpose(a_dst, (1, 0, 2)).astype(jnp.bfloat16)
    grid = (B, R)
    out = pl.pallas_call(
        _moe_gat_kernel,
        grid=grid,
        in_specs=[
            pl.BlockSpec((1, N, D), lambda b, r: (b, 0, 0)),
            pl.BlockSpec((1, 1, N, N), lambda b, r: (b, r, 0, 0)),
            pl.BlockSpec((D, E), lambda b, r: (0, 0)),
            pl.BlockSpec((1, E), lambda b, r: (0, 0)),
            pl.BlockSpec((1, E, D, D), lambda b, r: (r, 0, 0, 0)),
            pl.BlockSpec((1, 8 * E, D), lambda b, r: (r, 0, 0)),
            pl.BlockSpec((1, E, D), lambda b, r: (r, 0, 0)),
        ],
        out_specs=pl.BlockSpec((1, N, D), lambda b, r: (b, 0, 0)),
        out_shape=jax.ShapeDtypeStruct((B, N, D), jnp.float32),
        scratch_shapes=[
            pltpu.VMEM((N, E), jnp.float32),
        ],
        compiler_params=pltpu.CompilerParams(
            dimension_semantics=("arbitrary", "arbitrary"),
        ),
    )(x, adj, Wg, bg2, WrT, as8, adT)
    return out


# speedup vs baseline: 1.3706x; 1.0017x over previous
import jax
import jax.numpy as jnp
from jax import lax
from jax.experimental import pallas as pl
from jax.experimental.pallas import tpu as pltpu

B, N, D, R, E = 2, 1024, 128, 3, 8


def _moe_gat_kernel(x_ref, adj_ref, Wg_ref, bg_ref, Wr_ref, as_ref, ad_ref,
                    out_ref, gate_s):
    r_idx = pl.program_id(1)

    @pl.when(r_idx == 0)
    def _init():
        out_ref[...] = jnp.zeros_like(out_ref)
        xg = x_ref[0]
        gl = jnp.dot(xg, Wg_ref[...], preferred_element_type=jnp.float32)
        gl = gl + bg_ref[...]
        gl = gl - jnp.max(gl, axis=1, keepdims=True)
        gexp = jnp.exp(gl)
        gate_s[...] = gexp / jnp.sum(gexp, axis=1, keepdims=True)

    x_bf = x_ref[0].astype(jnp.bfloat16)
    maskbf = adj_ref[0, 0].astype(jnp.bfloat16)
    asrc_all = as_ref[0]
    adst_all = ad_ref[0]
    gate = gate_s[...]

    acc = jnp.zeros((N, D), jnp.float32)
    for e in range(E):
        W = Wr_ref[0, e]
        h = jnp.dot(x_bf, W,
                    preferred_element_type=jnp.float32).astype(jnp.bfloat16)

        asrc8 = asrc_all[8 * e:8 * e + 8, :]
        adst = adst_all[e:e + 1, :]
        es8 = lax.dot_general(h, asrc8, (((1,), (1,)), ((), ())),
                              preferred_element_type=jnp.float32)
        es = es8[:, 0:1]
        ed = lax.dot_general(adst, h, (((1,), (1,)), ((), ())),
                             preferred_element_type=jnp.float32)

        edmax = jnp.max(ed, axis=1, keepdims=True)
        se = es + edmax
        shift = jnp.maximum(se, 0.2 * se)
        a_pos = jnp.exp(se - shift).astype(jnp.bfloat16)
        a_neg = jnp.exp(0.2 * se - shift).astype(jnp.bfloat16)
        b_pos = jnp.exp(ed - edmax).astype(jnp.bfloat16)
        b_neg = jnp.exp(0.2 * (ed - edmax)).astype(jnp.bfloat16)

        pu = jnp.maximum(a_pos * b_pos, a_neg * b_neg)
        p = pu * maskbf

        denom = jnp.sum(p.astype(jnp.float32), axis=1,
                        keepdims=True)
        contrib = jnp.dot(p, h,
                          preferred_element_type=jnp.float32)

        ge = gate[:, e:e + 1]
        acc = acc + (ge / jnp.maximum(denom, jnp.float32(1e-30))) * contrib

    out_ref[0] = out_ref[0] + acc


def kernel(x, adj, Wg, bg, Wr, a_src, a_dst):
    bg2 = bg.reshape(1, E)
    WrT = jnp.transpose(Wr, (1, 0, 2, 3)).astype(jnp.bfloat16)
    asT = jnp.transpose(a_src, (1, 0, 2))
    as8 = jnp.zeros((R, 8 * E, D), jnp.float32)
    as8 = as8.at[:, ::8, :].set(asT).astype(jnp.bfloat16)
    adT = jnp.transpose(a_dst, (1, 0, 2)).astype(jnp.bfloat16)
    grid = (B, R)
    out = pl.pallas_call(
        _moe_gat_kernel,
        grid=grid,
        in_specs=[
            pl.BlockSpec((1, N, D), lambda b, r: (b, 0, 0)),
            pl.BlockSpec((1, 1, N, N), lambda b, r: (b, r, 0, 0)),
            pl.BlockSpec((D, E), lambda b, r: (0, 0)),
            pl.BlockSpec((1, E), lambda b, r: (0, 0)),
            pl.BlockSpec((1, E, D, D), lambda b, r: (r, 0, 0, 0)),
            pl.BlockSpec((1, 8 * E, D), lambda b, r: (r, 0, 0)),
            pl.BlockSpec((1, E, D), lambda b, r: (r, 0, 0)),
        ],
        out_specs=pl.BlockSpec((1, N, D), lambda b, r: (b, 0, 0)),
        out_shape=jax.ShapeDtypeStruct((B, N, D), jnp.float32),
        scratch_shapes=[
            pltpu.VMEM((N, E), jnp.float32),
        ],
        compiler_params=pltpu.CompilerParams(
            dimension_semantics=("parallel", "arbitrary"),
        ),
    )(x, adj, Wg, bg2, WrT, as8, adT)
    return out


# speedup vs baseline: 1.4228x; 1.0381x over previous
import jax
import jax.numpy as jnp
from jax import lax
from jax.experimental import pallas as pl
from jax.experimental.pallas import tpu as pltpu

B, N, D, R, E = 2, 1024, 128, 3, 8


def _moe_gat_kernel(x_ref, adj_ref, Wg_ref, bg_ref, Wr_ref, as_ref, ad_ref,
                    out_ref, gate_s):
    r_idx = pl.program_id(1)

    @pl.when(r_idx == 0)
    def _init():
        out_ref[...] = jnp.zeros_like(out_ref)
        xg = x_ref[0]
        gl = jnp.dot(xg, Wg_ref[...], preferred_element_type=jnp.float32)
        gl = gl + bg_ref[...]
        gl = gl - jnp.max(gl, axis=1, keepdims=True)
        gexp = jnp.exp(gl)
        gate_s[...] = gexp / jnp.sum(gexp, axis=1, keepdims=True)

    x_bf = x_ref[0].astype(jnp.bfloat16)
    maskbf = adj_ref[0, 0].astype(jnp.bfloat16)
    asrc_all = as_ref[0]
    adst_all = ad_ref[0]
    gate = gate_s[...]

    acc = jnp.zeros((N, D), jnp.float32)
    for e in range(E):
        W = Wr_ref[0, e]
        h = jnp.dot(x_bf, W,
                    preferred_element_type=jnp.float32).astype(jnp.bfloat16)

        asrc8 = asrc_all[8 * e:8 * e + 8, :]
        adst = adst_all[e:e + 1, :]
        es8 = lax.dot_general(h, asrc8, (((1,), (1,)), ((), ())),
                              preferred_element_type=jnp.float32)
        es = es8[:, 0:1]
        ed = lax.dot_general(adst, h, (((1,), (1,)), ((), ())),
                             preferred_element_type=jnp.float32)

        edmax = jnp.max(ed, axis=1, keepdims=True)
        se = es + edmax
        shift = jnp.maximum(se, 0.2 * se)
        a_pos = jnp.exp(se - shift).astype(jnp.bfloat16)
        a_neg = jnp.exp(0.2 * se - shift).astype(jnp.bfloat16)
        b_pos = jnp.exp(ed - edmax).astype(jnp.bfloat16)
        b_neg = jnp.exp(0.2 * (ed - edmax)).astype(jnp.bfloat16)

        pu = jnp.maximum(a_pos * b_pos, a_neg * b_neg)
        p = pu * maskbf

        ph = p[:, :N // 2] + p[:, N // 2:]
        pq = ph[:, :N // 4] + ph[:, N // 4:]
        denom = jnp.sum(pq.astype(jnp.float32), axis=1,
                        keepdims=True)
        contrib = jnp.dot(p, h,
                          preferred_element_type=jnp.float32)

        ge = gate[:, e:e + 1]
        acc = acc + (ge / jnp.maximum(denom, jnp.float32(1e-30))) * contrib

    out_ref[0] = out_ref[0] + acc


def kernel(x, adj, Wg, bg, Wr, a_src, a_dst):
    bg2 = bg.reshape(1, E)
    WrT = jnp.transpose(Wr, (1, 0, 2, 3)).astype(jnp.bfloat16)
    asT = jnp.transpose(a_src, (1, 0, 2))
    as8 = jnp.zeros((R, 8 * E, D), jnp.float32)
    as8 = as8.at[:, ::8, :].set(asT).astype(jnp.bfloat16)
    adT = jnp.transpose(a_dst, (1, 0, 2)).astype(jnp.bfloat16)
    grid = (B, R)
    out = pl.pallas_call(
        _moe_gat_kernel,
        grid=grid,
        in_specs=[
            pl.BlockSpec((1, N, D), lambda b, r: (b, 0, 0)),
            pl.BlockSpec((1, 1, N, N), lambda b, r: (b, r, 0, 0)),
            pl.BlockSpec((D, E), lambda b, r: (0, 0)),
            pl.BlockSpec((1, E), lambda b, r: (0, 0)),
            pl.BlockSpec((1, E, D, D), lambda b, r: (r, 0, 0, 0)),
            pl.BlockSpec((1, 8 * E, D), lambda b, r: (r, 0, 0)),
            pl.BlockSpec((1, E, D), lambda b, r: (r, 0, 0)),
        ],
        out_specs=pl.BlockSpec((1, N, D), lambda b, r: (b, 0, 0)),
        out_shape=jax.ShapeDtypeStruct((B, N, D), jnp.float32),
        scratch_shapes=[
            pltpu.VMEM((N, E), jnp.float32),
        ],
        compiler_params=pltpu.CompilerParams(
            dimension_semantics=("parallel", "arbitrary"),
        ),
    )(x, adj, Wg, bg2, WrT, as8, adT)
    return out


# speedup vs baseline: 1.4330x; 1.0072x over previous
import jax
import jax.numpy as jnp
from jax import lax
from jax.experimental import pallas as pl
from jax.experimental.pallas import tpu as pltpu

B, N, D, R, E = 2, 1024, 128, 3, 8


def _moe_gat_kernel(x_ref, adj_ref, Wg_ref, bg_ref, Wr_ref, as_ref, ad_ref,
                    out_ref, gate_s):
    r_idx = pl.program_id(1)

    @pl.when(r_idx == 0)
    def _init():
        out_ref[...] = jnp.zeros_like(out_ref)
        xg = x_ref[0]
        gl = jnp.dot(xg, Wg_ref[...], preferred_element_type=jnp.float32)
        gl = gl + bg_ref[...]
        gl = gl - jnp.max(gl, axis=1, keepdims=True)
        gexp = jnp.exp(gl)
        gate_s[...] = gexp / jnp.sum(gexp, axis=1, keepdims=True)

    x_bf = x_ref[0].astype(jnp.bfloat16)
    maskbf = adj_ref[0, 0].astype(jnp.bfloat16)
    asrc_all = as_ref[0]
    adst_all = ad_ref[0]
    gate = gate_s[...]

    acc = jnp.zeros((N, D), jnp.float32)
    for e in range(E):
        W = Wr_ref[0, e]
        h = jnp.dot(x_bf, W,
                    preferred_element_type=jnp.float32).astype(jnp.bfloat16)

        asrc8 = asrc_all[8 * e:8 * e + 8, :]
        adst = adst_all[e:e + 1, :]
        es8 = lax.dot_general(h, asrc8, (((1,), (1,)), ((), ())),
                              preferred_element_type=jnp.float32)
        es = es8[:, 0:1]
        ed = lax.dot_general(adst, h, (((1,), (1,)), ((), ())),
                             preferred_element_type=jnp.float32)

        edmax = jnp.max(ed, axis=1, keepdims=True)
        se = es + edmax
        shift = jnp.maximum(se, 0.2 * se)
        a_pos = jnp.exp(se - shift).astype(jnp.bfloat16)
        a_neg = jnp.exp(0.2 * se - shift).astype(jnp.bfloat16)
        b_pos = jnp.exp(ed - edmax).astype(jnp.bfloat16)
        b_neg = jnp.exp(0.2 * (ed - edmax)).astype(jnp.bfloat16)

        pu = jnp.maximum(a_pos * b_pos, a_neg * b_neg)
        p = pu * maskbf

        ph = p[:, :N // 2] + p[:, N // 2:]
        ph = ph[:, :N // 4] + ph[:, N // 4:]
        ph = ph[:, :N // 8] + ph[:, N // 8:]
        denom = jnp.sum(ph.astype(jnp.float32), axis=1,
                        keepdims=True)
        contrib = jnp.dot(p, h,
                          preferred_element_type=jnp.float32)

        ge = gate[:, e:e + 1]
        acc = acc + (ge / jnp.maximum(denom, jnp.float32(1e-30))) * contrib

    out_ref[0] = out_ref[0] + acc


def kernel(x, adj, Wg, bg, Wr, a_src, a_dst):
    bg2 = bg.reshape(1, E)
    WrT = jnp.transpose(Wr, (1, 0, 2, 3)).astype(jnp.bfloat16)
    asT = jnp.transpose(a_src, (1, 0, 2))
    as8 = jnp.zeros((R, 8 * E, D), jnp.float32)
    as8 = as8.at[:, ::8, :].set(asT).astype(jnp.bfloat16)
    adT = jnp.transpose(a_dst, (1, 0, 2)).astype(jnp.bfloat16)
    grid = (B, R)
    out = pl.pallas_call(
        _moe_gat_kernel,
        grid=grid,
        in_specs=[
            pl.BlockSpec((1, N, D), lambda b, r: (b, 0, 0)),
            pl.BlockSpec((1, 1, N, N), lambda b, r: (b, r, 0, 0)),
            pl.BlockSpec((D, E), lambda b, r: (0, 0)),
            pl.BlockSpec((1, E), lambda b, r: (0, 0)),
            pl.BlockSpec((1, E, D, D), lambda b, r: (r, 0, 0, 0)),
            pl.BlockSpec((1, 8 * E, D), lambda b, r: (r, 0, 0)),
            pl.BlockSpec((1, E, D), lambda b, r: (r, 0, 0)),
        ],
        out_specs=pl.BlockSpec((1, N, D), lambda b, r: (b, 0, 0)),
        out_shape=jax.ShapeDtypeStruct((B, N, D), jnp.float32),
        scratch_shapes=[
            pltpu.VMEM((N, E), jnp.float32),
        ],
        compiler_params=pltpu.CompilerParams(
            dimension_semantics=("parallel", "arbitrary"),
        ),
    )(x, adj, Wg, bg2, WrT, as8, adT)
    return out
